# Initial kernel scaffold; baseline (speedup 1.0000x reference)
import jax
import jax.numpy as jnp
from jax.experimental import pallas as pl


def kernel(x, edge_index, W1, b1, W2, b2):
    raise NotImplementedError("write your pallas kernel here")


# speedup vs baseline: 151.3781x; 151.3781x over previous
import functools

import jax
import jax.numpy as jnp
from jax import lax
from jax.experimental import pallas as pl
from jax.experimental.pallas import tpu as pltpu
from jax.experimental.pallas import tpu_sc as plsc

N = 100000
E = 6400000
F = 8
NPAD = 100352
STRIPE = NPAD // 16
NC = 2
NS = 16
NW = NC * NS
EPW = E // NW
CHUNK1 = 2000
NITER1 = EPW // CHUNK1
CHUNK = 8000
NITER = EPW // CHUNK

_mesh = plsc.VectorSubcoreMesh(core_axis_name="c", subcore_axis_name="s")


def _make_deg_kernel():
    @functools.partial(
        pl.kernel,
        mesh=_mesh,
        compiler_params=pltpu.CompilerParams(use_tc_tiling_on_sc=False),
        out_type=jax.ShapeDtypeStruct((2 * NPAD,), jnp.float32),
        scratch_types=[
            pltpu.VMEM((CHUNK,), jnp.int32),
            pltpu.VMEM((CHUNK,), jnp.float32),
            pltpu.VMEM_SHARED((NPAD,), jnp.float32),
        ],
    )
    def deg_kernel(dst_hbm, zeros_hbm, ones_hbm, out_hbm, idx_v, ones_v, acc_sh):
        cid = lax.axis_index("c")
        sid = lax.axis_index("s")
        wid = cid * NS + sid
        stripe = pl.ds(sid * STRIPE, STRIPE)
        pltpu.sync_copy(zeros_hbm.at[stripe], acc_sh.at[stripe])
        pltpu.sync_copy(ones_hbm, ones_v)
        plsc.subcore_barrier()
        base = wid * EPW

        def body(i, carry):
            off = base + i * CHUNK
            pltpu.sync_copy(dst_hbm.at[pl.ds(off, CHUNK)], idx_v)
            pltpu.sync_copy(ones_v, acc_sh.at[idx_v], add=True)
            return carry

        lax.fori_loop(0, NITER, body, 0)
        plsc.subcore_barrier()
        pltpu.sync_copy(acc_sh.at[stripe],
                        out_hbm.at[pl.ds(cid * NPAD + sid * STRIPE, STRIPE)])

    return deg_kernel


def _make_prop1_kernel():
    @functools.partial(
        pl.kernel,
        mesh=_mesh,
        compiler_params=pltpu.CompilerParams(use_tc_tiling_on_sc=False),
        out_type=jax.ShapeDtypeStruct((2 * NPAD, F), jnp.float32),
        scratch_types=[
            pltpu.VMEM((CHUNK1,), jnp.int32),
            pltpu.VMEM((CHUNK1,), jnp.int32),
            pltpu.VMEM((CHUNK1, F), jnp.float32),
            pltpu.VMEM_SHARED((NPAD, F), jnp.float32),
            pltpu.VMEM_SHARED((NPAD, F), jnp.float32),
            pltpu.SemaphoreType.DMA,
        ],
    )
    def prop1_kernel(src_hbm, dst_hbm, y_hbm, zeros_hbm, out_hbm,
                     sidx_v, didx_v, rows_v, acc_sh, y_sh, sem):
        cid = lax.axis_index("c")
        sid = lax.axis_index("s")
        wid = cid * NS + sid
        stripe = pl.ds(sid * STRIPE, STRIPE)
        pltpu.sync_copy(zeros_hbm.at[stripe], acc_sh.at[stripe])
        pltpu.sync_copy(y_hbm.at[stripe], y_sh.at[stripe])
        plsc.subcore_barrier()
        base = wid * EPW

        def body(i, carry):
            off = base + i * CHUNK1
            pltpu.sync_copy(src_hbm.at[pl.ds(off, CHUNK1)], sidx_v)
            pltpu.sync_copy(dst_hbm.at[pl.ds(off, CHUNK1)], didx_v)
            pltpu.async_copy(y_sh.at[sidx_v], rows_v, sem).wait()
            pltpu.sync_copy(rows_v, acc_sh.at[didx_v], add=True)
            return carry

        lax.fori_loop(0, NITER1, body, 0)
        plsc.subcore_barrier()
        pltpu.sync_copy(acc_sh.at[stripe],
                        out_hbm.at[pl.ds(cid * NPAD + sid * STRIPE, STRIPE)])

    return prop1_kernel


def _make_prop2_kernel():
    @functools.partial(
        pl.kernel,
        mesh=_mesh,
        compiler_params=pltpu.CompilerParams(use_tc_tiling_on_sc=False),
        out_type=jax.ShapeDtypeStruct((2 * NPAD,), jnp.float32),
        scratch_types=[
            pltpu.VMEM((CHUNK,), jnp.int32),
            pltpu.VMEM((CHUNK,), jnp.int32),
            pltpu.VMEM((CHUNK,), jnp.float32),
            pltpu.VMEM_SHARED((NPAD,), jnp.float32),
            pltpu.VMEM_SHARED((NPAD,), jnp.float32),
            pltpu.SemaphoreType.DMA,
        ],
    )
    def prop2_kernel(src_hbm, dst_hbm, yh_hbm, zeros_hbm, out_hbm,
                     sidx_v, didx_v, vals_v, yh_sh, acc_sh, sem):
        cid = lax.axis_index("c")
        sid = lax.axis_index("s")
        wid = cid * NS + sid
        stripe = pl.ds(sid * STRIPE, STRIPE)
        pltpu.sync_copy(zeros_hbm.at[stripe], acc_sh.at[stripe])
        pltpu.sync_copy(yh_hbm.at[stripe], yh_sh.at[stripe])
        plsc.subcore_barrier()
        base = wid * EPW

        def body(i, carry):
            off = base + i * CHUNK
            pltpu.sync_copy(src_hbm.at[pl.ds(off, CHUNK)], sidx_v)
            pltpu.sync_copy(dst_hbm.at[pl.ds(off, CHUNK)], didx_v)
            pltpu.async_copy(yh_sh.at[sidx_v], vals_v, sem).wait()
            pltpu.sync_copy(vals_v, acc_sh.at[didx_v], add=True)
            return carry

        lax.fori_loop(0, NITER, body, 0)
        plsc.subcore_barrier()
        pltpu.sync_copy(acc_sh.at[stripe],
                        out_hbm.at[pl.ds(cid * NPAD + sid * STRIPE, STRIPE)])

    return prop2_kernel


_BLK = 1024
_GRID = NPAD // _BLK


def _tc1_body(d_ref, x_ref, y_ref):
    deg = d_ref[0, :] + d_ref[1, :] + 1.0
    dinv = lax.rsqrt(deg)
    y_ref[...] = x_ref[...] * dinv[:, None]


def _tc1(d_part, x_pad):
    return pl.pallas_call(
        _tc1_body,
        grid=(_GRID,),
        in_specs=[
            pl.BlockSpec((2, _BLK), lambda i: (0, i)),
            pl.BlockSpec((_BLK, F), lambda i: (i, 0)),
        ],
        out_specs=pl.BlockSpec((_BLK, F), lambda i: (i, 0)),
        out_shape=jax.ShapeDtypeStruct((NPAD, F), jnp.float32),
    )(d_part, x_pad)


def _tc2_body(p_ref, y_ref, w1_ref, b1_ref, w2_ref, yh_ref):
    y = y_ref[...]
    s = p_ref[0] + p_ref[1] + y
    dinv = y[:, 5]
    pt = s[:, :5] * dinv[:, None]
    h = pt @ w1_ref[...] + b1_ref[...]
    h = jnp.maximum(h, 0.0)
    hw = jnp.sum(h * w2_ref[...], axis=1)
    yh_ref[...] = (hw * dinv)[:, None]


def _tc2(p_part, y_pad, W1, b1, W2):
    return pl.pallas_call(
        _tc2_body,
        grid=(_GRID,),
        in_specs=[
            pl.BlockSpec((2, _BLK, F), lambda i: (0, i, 0)),
            pl.BlockSpec((_BLK, F), lambda i: (i, 0)),
            pl.BlockSpec((5, 64), lambda i: (0, 0)),
            pl.BlockSpec((1, 64), lambda i: (0, 0)),
            pl.BlockSpec((1, 64), lambda i: (0, 0)),
        ],
        out_specs=pl.BlockSpec((_BLK, 1), lambda i: (i, 0)),
        out_shape=jax.ShapeDtypeStruct((NPAD, 1), jnp.float32),
    )(p_part, y_pad, W1, b1, W2)


def _tc3_body(z_ref, yh_ref, y_ref, b2_ref, o_ref):
    zsum = z_ref[0] + z_ref[1] + yh_ref[:, 0]
    dinv = y_ref[:, 5]
    o_ref[...] = (zsum * dinv + b2_ref[0, 0])[:, None]


def _tc3(z_part, yh, y_pad, b2):
    return pl.pallas_call(
        _tc3_body,
        grid=(_GRID,),
        in_specs=[
            pl.BlockSpec((2, _BLK), lambda i: (0, i)),
            pl.BlockSpec((_BLK, 1), lambda i: (i, 0)),
            pl.BlockSpec((_BLK, F), lambda i: (i, 0)),
            pl.BlockSpec((1, 1), lambda i: (0, 0)),
        ],
        out_specs=pl.BlockSpec((_BLK, 1), lambda i: (i, 0)),
        out_shape=jax.ShapeDtypeStruct((NPAD, 1), jnp.float32),
    )(z_part, yh, y_pad, b2)


_deg = _make_deg_kernel()
_prop1 = _make_prop1_kernel()
_prop2 = _make_prop2_kernel()


def kernel(x, edge_index, W1, b1, W2, b2):
    x_pad = jnp.zeros((NPAD, F), jnp.float32)
    x_pad = x_pad.at[:N, :5].set(x)
    x_pad = x_pad.at[:, 5].set(1.0)

    zeros_n = jnp.zeros((NPAD,), jnp.float32)
    zeros_nf = jnp.zeros((NPAD, F), jnp.float32)
    ones_c = jnp.ones((CHUNK,), jnp.float32)

    src = edge_index[0]
    dst = edge_index[1]
    d_flat = _deg(dst, zeros_n, ones_c)
    d_part = d_flat.reshape(2, NPAD)
    y_pad = _tc1(d_part, x_pad)

    p_flat = _prop1(src, dst, y_pad, zeros_nf)
    p_part = p_flat.reshape(2, NPAD, F)
    yh = _tc2(p_part, y_pad, W1, b1.reshape(1, 64), W2.reshape(1, 64))

    z_flat = _prop2(src, dst, yh.reshape(NPAD), zeros_n)
    z_part = z_flat.reshape(2, NPAD)
    out = _tc3(z_part, yh, y_pad, b2.reshape(1, 1))
    return out[:N]


# speedup vs baseline: 191.8877x; 1.2676x over previous
import functools

import jax
import jax.numpy as jnp
from jax import lax
from jax.experimental import pallas as pl
from jax.experimental.pallas import tpu as pltpu
from jax.experimental.pallas import tpu_sc as plsc

N = 100000
E = 6400000
F = 8
NPAD = 106496
STRIPE = NPAD // 16
NC = 2
NS = 16
NW = NC * NS
EPW = E // NW
CHUNK1 = 2000
NITER1 = EPW // CHUNK1
CHUNK = 8000
NITER = EPW // CHUNK

_mesh = plsc.VectorSubcoreMesh(core_axis_name="c", subcore_axis_name="s")


def _make_deg_kernel():
    @functools.partial(
        pl.kernel,
        mesh=_mesh,
        compiler_params=pltpu.CompilerParams(use_tc_tiling_on_sc=False),
        out_type=jax.ShapeDtypeStruct((2 * NPAD,), jnp.float32),
        scratch_types=[
            pltpu.VMEM((CHUNK,), jnp.int32),
            pltpu.VMEM((CHUNK,), jnp.float32),
            pltpu.VMEM_SHARED((NPAD,), jnp.float32),
        ],
    )
    def deg_kernel(dst_hbm, zeros_hbm, ones_hbm, out_hbm, idx_v, ones_v, acc_sh):
        cid = lax.axis_index("c")
        sid = lax.axis_index("s")
        wid = cid * NS + sid
        stripe = pl.ds(sid * STRIPE, STRIPE)
        pltpu.sync_copy(zeros_hbm.at[stripe], acc_sh.at[stripe])
        pltpu.sync_copy(ones_hbm, ones_v)
        plsc.subcore_barrier()
        base = wid * EPW

        def body(i, carry):
            off = base + i * CHUNK
            pltpu.sync_copy(dst_hbm.at[pl.ds(off, CHUNK)], idx_v)
            pltpu.sync_copy(ones_v, acc_sh.at[idx_v], add=True)
            return carry

        lax.fori_loop(0, NITER, body, 0)
        plsc.subcore_barrier()
        pltpu.sync_copy(acc_sh.at[stripe],
                        out_hbm.at[pl.ds(cid * NPAD + sid * STRIPE, STRIPE)])

    return deg_kernel


def _make_prop1_kernel():
    @functools.partial(
        pl.kernel,
        mesh=_mesh,
        compiler_params=pltpu.CompilerParams(use_tc_tiling_on_sc=False),
        out_type=jax.ShapeDtypeStruct((2 * NPAD, F), jnp.float32),
        scratch_types=[
            pltpu.VMEM((CHUNK1,), jnp.int32),
            pltpu.VMEM((CHUNK1,), jnp.int32),
            pltpu.VMEM((CHUNK1, F), jnp.float32),
            pltpu.VMEM_SHARED((NPAD, F), jnp.float32),
            pltpu.VMEM_SHARED((NPAD, F), jnp.float32),
            pltpu.SemaphoreType.DMA,
        ],
    )
    def prop1_kernel(src_hbm, dst_hbm, y_hbm, zeros_hbm, out_hbm,
                     sidx_v, didx_v, rows_v, acc_sh, y_sh, sem):
        cid = lax.axis_index("c")
        sid = lax.axis_index("s")
        wid = cid * NS + sid
        stripe = pl.ds(sid * STRIPE, STRIPE)
        pltpu.sync_copy(zeros_hbm.at[stripe], acc_sh.at[stripe])
        pltpu.sync_copy(y_hbm.at[stripe], y_sh.at[stripe])
        plsc.subcore_barrier()
        base = wid * EPW

        def body(i, carry):
            off = base + i * CHUNK1
            pltpu.sync_copy(src_hbm.at[pl.ds(off, CHUNK1)], sidx_v)
            pltpu.sync_copy(dst_hbm.at[pl.ds(off, CHUNK1)], didx_v)
            pltpu.async_copy(y_sh.at[sidx_v], rows_v, sem).wait()
            pltpu.sync_copy(rows_v, acc_sh.at[didx_v], add=True)
            return carry

        lax.fori_loop(0, NITER1, body, 0)
        plsc.subcore_barrier()
        pltpu.sync_copy(acc_sh.at[stripe],
                        out_hbm.at[pl.ds(cid * NPAD + sid * STRIPE, STRIPE)])

    return prop1_kernel


def _make_prop2_kernel():
    @functools.partial(
        pl.kernel,
        mesh=_mesh,
        compiler_params=pltpu.CompilerParams(use_tc_tiling_on_sc=False),
        out_type=jax.ShapeDtypeStruct((2 * NPAD,), jnp.float32),
        scratch_types=[
            pltpu.VMEM((CHUNK,), jnp.int32),
            pltpu.VMEM((CHUNK,), jnp.int32),
            pltpu.VMEM((CHUNK,), jnp.float32),
            pltpu.VMEM_SHARED((NPAD,), jnp.float32),
            pltpu.VMEM_SHARED((NPAD,), jnp.float32),
            pltpu.SemaphoreType.DMA,
        ],
    )
    def prop2_kernel(src_hbm, dst_hbm, yh_hbm, zeros_hbm, out_hbm,
                     sidx_v, didx_v, vals_v, yh_sh, acc_sh, sem):
        cid = lax.axis_index("c")
        sid = lax.axis_index("s")
        wid = cid * NS + sid
        stripe = pl.ds(sid * STRIPE, STRIPE)
        pltpu.sync_copy(zeros_hbm.at[stripe], acc_sh.at[stripe])
        pltpu.sync_copy(yh_hbm.at[stripe], yh_sh.at[stripe])
        plsc.subcore_barrier()
        base = wid * EPW

        def body(i, carry):
            off = base + i * CHUNK
            pltpu.sync_copy(src_hbm.at[pl.ds(off, CHUNK)], sidx_v)
            pltpu.sync_copy(dst_hbm.at[pl.ds(off, CHUNK)], didx_v)
            pltpu.async_copy(yh_sh.at[sidx_v], vals_v, sem).wait()
            pltpu.sync_copy(vals_v, acc_sh.at[didx_v], add=True)
            return carry

        lax.fori_loop(0, NITER, body, 0)
        plsc.subcore_barrier()
        pltpu.sync_copy(acc_sh.at[stripe],
                        out_hbm.at[pl.ds(cid * NPAD + sid * STRIPE, STRIPE)])

    return prop2_kernel


_BLK = 8192
_GRID = NPAD // _BLK


def _tc1_body(d_ref, x_ref, y_ref):
    deg = d_ref[0, :] + d_ref[1, :] + 1.0
    dinv = lax.rsqrt(deg)
    col = dinv[:, None]
    z = jnp.zeros_like(col)
    y_ref[...] = jnp.concatenate([x_ref[...] * col, col, z, z], axis=1)


def _tc1(d_part, x):
    return pl.pallas_call(
        _tc1_body,
        grid=(_GRID,),
        in_specs=[
            pl.BlockSpec((2, _BLK), lambda i: (0, i)),
            pl.BlockSpec((_BLK, 5), lambda i: (i, 0)),
        ],
        out_specs=pl.BlockSpec((_BLK, F), lambda i: (i, 0)),
        out_shape=jax.ShapeDtypeStruct((NPAD, F), jnp.float32),
    )(d_part, x)


def _tc2_body(p0_ref, p1_ref, y_ref, w1_ref, b1_ref, w2_ref, yh_ref):
    y = y_ref[...]
    s = p0_ref[...] + p1_ref[...] + y
    dinv = y[:, 5]
    pt = s[:, :5] * dinv[:, None]
    h = pt @ w1_ref[...] + b1_ref[...]
    h = jnp.maximum(h, 0.0)
    hw = jnp.sum(h * w2_ref[...], axis=1)
    yh_ref[...] = (hw * dinv)[:, None]


def _tc2(p_flat, y_pad, W1, b1, W2):
    nb = NPAD // _BLK
    return pl.pallas_call(
        _tc2_body,
        grid=(_GRID,),
        in_specs=[
            pl.BlockSpec((_BLK, F), lambda i: (i, 0)),
            pl.BlockSpec((_BLK, F), lambda i: (i + nb, 0)),
            pl.BlockSpec((_BLK, F), lambda i: (i, 0)),
            pl.BlockSpec((5, 64), lambda i: (0, 0)),
            pl.BlockSpec((1, 64), lambda i: (0, 0)),
            pl.BlockSpec((1, 64), lambda i: (0, 0)),
        ],
        out_specs=pl.BlockSpec((_BLK, 1), lambda i: (i, 0)),
        out_shape=jax.ShapeDtypeStruct((NPAD, 1), jnp.float32),
    )(p_flat, p_flat, y_pad, W1, b1, W2)


def _tc3_body(z_ref, yh_ref, y_ref, b2_ref, o_ref):
    zsum = z_ref[0] + z_ref[1] + yh_ref[:, 0]
    dinv = y_ref[:, 5]
    o_ref[...] = (zsum * dinv + b2_ref[0, 0])[:, None]


def _tc3(z_part, yh, y_pad, b2):
    return pl.pallas_call(
        _tc3_body,
        grid=(_GRID,),
        in_specs=[
            pl.BlockSpec((2, _BLK), lambda i: (0, i)),
            pl.BlockSpec((_BLK, 1), lambda i: (i, 0)),
            pl.BlockSpec((_BLK, F), lambda i: (i, 0)),
            pl.BlockSpec((1, 1), lambda i: (0, 0)),
        ],
        out_specs=pl.BlockSpec((_BLK, 1), lambda i: (i, 0)),
        out_shape=jax.ShapeDtypeStruct((N, 1), jnp.float32),
    )(z_part, yh, y_pad, b2)


_deg = _make_deg_kernel()
_prop1 = _make_prop1_kernel()
_prop2 = _make_prop2_kernel()


def kernel(x, edge_index, W1, b1, W2, b2):
    zeros_n = jnp.zeros((NPAD,), jnp.float32)
    zeros_nf = jnp.zeros((NPAD, F), jnp.float32)
    ones_c = jnp.ones((CHUNK,), jnp.float32)

    src = edge_index[0]
    dst = edge_index[1]
    d_flat = _deg(dst, zeros_n, ones_c)
    d_part = d_flat.reshape(2, NPAD)
    y_pad = _tc1(d_part, x)

    p_flat = _prop1(src, dst, y_pad, zeros_nf)
    yh = _tc2(p_flat, y_pad, W1, b1.reshape(1, 64), W2.reshape(1, 64))

    z_flat = _prop2(src, dst, yh.reshape(NPAD), zeros_n)
    z_part = z_flat.reshape(2, NPAD)
    return _tc3(z_part, yh, y_pad, b2.reshape(1, 1))
